# baseline (speedup 1.0000x reference)
import functools

import jax
import jax.numpy as jnp
from jax import lax
from jax.experimental import pallas as pl
from jax.experimental.pallas import tpu as pltpu
from jax.experimental.pallas import tpu_sc as plsc

N_FIELDS = 26
VOCAB = 100000
EMBED_DIM = 16
BATCH = 16384

NUM_CORES = 2
NUM_SUBCORES = 16
NUM_WORKERS = NUM_CORES * NUM_SUBCORES

TOTAL_ROWS = BATCH * N_FIELDS
ROWS_PER_WORKER = TOTAL_ROWS // NUM_WORKERS
BLOCK = 1664
BLOCKS_PER_WORKER = ROWS_PER_WORKER // BLOCK
LANES = 16


def _body(x_hbm, offs_hbm, tab_hbm, out_hbm, idx_v, offs_v, rows_v, sem):
    wid = lax.axis_index("s") * NUM_CORES + lax.axis_index("c")
    worker_base = wid * ROWS_PER_WORKER

    pltpu.sync_copy(offs_hbm, offs_v)

    def block_step(blk, _):
        base = worker_base + blk * BLOCK
        pltpu.sync_copy(x_hbm.at[pl.ds(base, BLOCK)], idx_v)

        def add_step(j, _):
            sl = pl.ds(j * LANES, LANES)
            idx_v[sl] = idx_v[sl] + offs_v[sl]
            return ()

        lax.fori_loop(0, BLOCK // LANES, add_step, ())
        pltpu.async_copy(tab_hbm.at[idx_v], rows_v, sem).wait()
        pltpu.sync_copy(rows_v, out_hbm.at[pl.ds(base, BLOCK)])
        return ()

    lax.fori_loop(0, BLOCKS_PER_WORKER, block_step, ())


@functools.partial(jax.jit, static_argnames=())
def kernel(x, tables):
    x_flat = x.astype(jnp.int32).reshape(TOTAL_ROWS)
    tab_flat = tables.reshape(N_FIELDS * VOCAB, EMBED_DIM)
    offs = jnp.tile(jnp.arange(N_FIELDS, dtype=jnp.int32) * VOCAB,
                    BLOCK // N_FIELDS)

    mesh = plsc.VectorSubcoreMesh(core_axis_name="c", subcore_axis_name="s")
    out = pl.kernel(
        _body,
        out_type=jax.ShapeDtypeStruct((TOTAL_ROWS, EMBED_DIM), jnp.float32),
        mesh=mesh,
        scratch_types=[
            pltpu.VMEM((BLOCK,), jnp.int32),
            pltpu.VMEM((BLOCK,), jnp.int32),
            pltpu.VMEM((BLOCK, EMBED_DIM), jnp.float32),
            pltpu.SemaphoreType.DMA,
        ],
        compiler_params=pltpu.CompilerParams(use_tc_tiling_on_sc=False),
    )(x_flat, offs, tab_flat)
    return out.reshape(BATCH, N_FIELDS, EMBED_DIM)


# speedup vs baseline: 1.1468x; 1.1468x over previous
import functools

import jax
import jax.numpy as jnp
from jax import lax
from jax.experimental import pallas as pl
from jax.experimental.pallas import tpu as pltpu
from jax.experimental.pallas import tpu_sc as plsc

N_FIELDS = 26
VOCAB = 100000
EMBED_DIM = 16
BATCH = 16384

NUM_CORES = 2
NUM_SUBCORES = 16
NUM_WORKERS = NUM_CORES * NUM_SUBCORES

BATCH_PER_WORKER = BATCH // NUM_WORKERS
LANES = 16


def _body(xt_hbm, tab_hbm, out_hbm, idx_v, rows_v, sem):
    wid = lax.axis_index("s") * NUM_CORES + lax.axis_index("c")
    b0 = wid * BATCH_PER_WORKER

    for f in range(N_FIELDS):
        pltpu.sync_copy(xt_hbm.at[f, pl.ds(b0, BATCH_PER_WORKER)], idx_v)
        pltpu.async_copy(tab_hbm.at[f].at[idx_v], rows_v, sem).wait()
        pltpu.sync_copy(rows_v, out_hbm.at[pl.ds(b0, BATCH_PER_WORKER), f])


@functools.partial(jax.jit, static_argnames=())
def kernel(x, tables):
    xt = x.astype(jnp.int32).T

    mesh = plsc.VectorSubcoreMesh(core_axis_name="c", subcore_axis_name="s")
    out = pl.kernel(
        _body,
        out_type=jax.ShapeDtypeStruct((BATCH, N_FIELDS, EMBED_DIM),
                                      jnp.float32),
        mesh=mesh,
        scratch_types=[
            pltpu.VMEM((BATCH_PER_WORKER,), jnp.int32),
            pltpu.VMEM((BATCH_PER_WORKER, EMBED_DIM), jnp.float32),
            pltpu.SemaphoreType.DMA,
        ],
        compiler_params=pltpu.CompilerParams(use_tc_tiling_on_sc=False),
    )(xt, tables)
    return out


# speedup vs baseline: 1.2151x; 1.0596x over previous
import functools

import jax
import jax.numpy as jnp
from jax import lax
from jax.experimental import pallas as pl
from jax.experimental.pallas import tpu as pltpu
from jax.experimental.pallas import tpu_sc as plsc

N_FIELDS = 26
VOCAB = 100000
EMBED_DIM = 16
BATCH = 16384

NUM_CORES = 2
NUM_SUBCORES = 16
NUM_WORKERS = NUM_CORES * NUM_SUBCORES

BATCH_PER_WORKER = BATCH // NUM_WORKERS
BT_PER_WORKER = BATCH_PER_WORKER // 128
GROUPS = N_FIELDS * VOCAB // 8
GPF = VOCAB // 8
LANES = 16


def _body(xt_hbm, tab_hbm, out_hbm, idx_v, gidx_v, grp_v, out_v, sem):
    wid = lax.axis_index("s") * NUM_CORES + lax.axis_index("c")
    b0 = wid * BATCH_PER_WORKER
    bt0 = wid * BT_PER_WORKER
    lane = lax.iota(jnp.int32, LANES)

    def field_step(f, _):
        pltpu.sync_copy(xt_hbm.at[pl.ds(f * BATCH + b0, BATCH_PER_WORKER)],
                        idx_v)

        def gid_step(j, _):
            sl = pl.ds(j * LANES, LANES)
            gidx_v[sl] = (idx_v[sl] >> 3) + f * GPF
            return ()

        lax.fori_loop(0, BATCH_PER_WORKER // LANES, gid_step, ())
        pltpu.async_copy(tab_hbm.at[gidx_v], grp_v, sem).wait()

        def ex_step(i, _):
            row = i * LANES + lane
            bt = i >> 3
            sl = pl.ds(i * LANES, LANES)
            colbase = (idx_v[sl] & 7) * LANES
            for et in range(2):
                for e in range(8):
                    val = plsc.load_gather(grp_v, [row, colbase + (et * 8 + e)])
                    out_v[et, bt, e, pl.ds((i & 7) * LANES, LANES)] = val
            return ()

        lax.fori_loop(0, BATCH_PER_WORKER // LANES, ex_step, ())
        for et in range(2):
            pltpu.sync_copy(out_v.at[et],
                            out_hbm.at[f, et, pl.ds(bt0, BT_PER_WORKER)])
        return ()

    lax.fori_loop(0, N_FIELDS, field_step, ())


@functools.partial(jax.jit, static_argnames=())
def kernel(x, tables):
    xt_flat = x.astype(jnp.int32).T.reshape(N_FIELDS * BATCH)
    tab_grp = tables.reshape(GROUPS, 8 * EMBED_DIM)

    mesh = plsc.VectorSubcoreMesh(core_axis_name="c", subcore_axis_name="s")
    out5 = pl.kernel(
        _body,
        out_type=jax.ShapeDtypeStruct(
            (N_FIELDS, 2, BATCH // 128, 8, 128), jnp.float32),
        mesh=mesh,
        scratch_types=[
            pltpu.VMEM((BATCH_PER_WORKER,), jnp.int32),
            pltpu.VMEM((BATCH_PER_WORKER,), jnp.int32),
            pltpu.VMEM((BATCH_PER_WORKER, 8 * EMBED_DIM), jnp.float32),
            pltpu.VMEM((2, BT_PER_WORKER, 8, 128), jnp.float32),
            pltpu.SemaphoreType.DMA,
        ],
        compiler_params=pltpu.CompilerParams(use_tc_tiling_on_sc=True,
                                             needs_layout_passes=False),
    )(xt_flat, tab_grp)
    return out5.transpose(2, 4, 0, 1, 3).reshape(BATCH, N_FIELDS, EMBED_DIM)


# speedup vs baseline: 3.2402x; 2.6665x over previous
import functools

import jax
import jax.numpy as jnp
from jax import lax
from jax.experimental import pallas as pl
from jax.experimental.pallas import tpu as pltpu
from jax.experimental.pallas import tpu_sc as plsc

N_FIELDS = 26
VOCAB = 100000
EMBED_DIM = 16
BATCH = 16384

NUM_CORES = 2
NUM_SUBCORES = 16
NUM_WORKERS = NUM_CORES * NUM_SUBCORES
PAIRS = N_FIELDS * EMBED_DIM
PAIRS_PER_WORKER = PAIRS // NUM_WORKERS
IDX_CHUNK = 4096
LANES = 16


def _body(xt_hbm, tab_hbm, out_hbm, row_v, idx_v, ov_v, sem):
    wid = lax.axis_index("s") * NUM_CORES + lax.axis_index("c")
    p0 = wid * PAIRS_PER_WORKER

    def pair_step(k, _):
        p = p0 + k
        f = p >> 4
        e = p & 15
        pltpu.sync_copy(tab_hbm.at[f, e], row_v)

        def chunk_step(c, _):
            pltpu.sync_copy(
                xt_hbm.at[pl.ds(f * BATCH + c * IDX_CHUNK, IDX_CHUNK)], idx_v)

            def gat_step(i, _):
                val = plsc.load_gather(row_v, [idx_v[pl.ds(i * LANES, LANES)]])
                pos = c * IDX_CHUNK + i * LANES
                ov_v[pos >> 7, pl.ds(pos & 127, LANES)] = val
                return ()

            lax.fori_loop(0, IDX_CHUNK // LANES, gat_step, ())
            return ()

        lax.fori_loop(0, BATCH // IDX_CHUNK, chunk_step, ())
        pltpu.sync_copy(ov_v, out_hbm.at[f, e >> 3, :, e & 7, :])
        return ()

    lax.fori_loop(0, PAIRS_PER_WORKER, pair_step, ())


@functools.partial(jax.jit, static_argnames=())
def kernel(x, tables):
    xt_flat = x.astype(jnp.int32).T.reshape(N_FIELDS * BATCH)
    tab_t = tables.transpose(0, 2, 1)

    mesh = plsc.VectorSubcoreMesh(core_axis_name="c", subcore_axis_name="s")
    out5 = pl.kernel(
        _body,
        out_type=jax.ShapeDtypeStruct(
            (N_FIELDS, 2, BATCH // 128, 8, 128), jnp.float32),
        mesh=mesh,
        scratch_types=[
            pltpu.VMEM((VOCAB,), jnp.float32),
            pltpu.VMEM((IDX_CHUNK,), jnp.int32),
            pltpu.VMEM((BATCH // 128, 128), jnp.float32),
            pltpu.SemaphoreType.DMA,
        ],
        compiler_params=pltpu.CompilerParams(use_tc_tiling_on_sc=False,
                                             needs_layout_passes=False),
    )(xt_flat, tab_t)
    return out5.transpose(2, 4, 0, 1, 3).reshape(BATCH, N_FIELDS, EMBED_DIM)


# speedup vs baseline: 6.2463x; 1.9278x over previous
import functools

import jax
import jax.numpy as jnp
from jax import lax
from jax.experimental import pallas as pl
from jax.experimental.pallas import tpu as pltpu
from jax.experimental.pallas import tpu_sc as plsc

N_FIELDS = 26
VOCAB = 100000
EMBED_DIM = 16
BATCH = 16384

NUM_CORES = 2
NUM_SUBCORES = 16
NUM_WORKERS = NUM_CORES * NUM_SUBCORES
PAIRS = N_FIELDS * EMBED_DIM
PAIRS_PER_WORKER = PAIRS // NUM_WORKERS
IDX_CHUNK = 4096
LANES = 16


def _body(xt_hbm, tab_hbm, out_hbm, row_v, idx_v, ov_v, sem):
    wid = lax.axis_index("s") * NUM_CORES + lax.axis_index("c")
    p0 = wid * PAIRS_PER_WORKER

    def pair_step(k, _):
        p = p0 + k
        f = p >> 4
        e = p & 15
        pltpu.sync_copy(tab_hbm.at[f, e], row_v)

        def chunk_step(c, _):
            pltpu.sync_copy(
                xt_hbm.at[pl.ds(f * BATCH + c * IDX_CHUNK, IDX_CHUNK)], idx_v)

            def gat_step(i, _):
                val = plsc.load_gather(row_v, [idx_v[pl.ds(i * LANES, LANES)]])
                pos = c * IDX_CHUNK + i * LANES
                ov_v[pos >> 7, pl.ds(pos & 127, LANES)] = val
                return ()

            lax.fori_loop(0, IDX_CHUNK // LANES, gat_step, ())
            return ()

        lax.fori_loop(0, BATCH // IDX_CHUNK, chunk_step, ())
        pltpu.sync_copy(ov_v, out_hbm.at[f, e >> 3, :, e & 7, :])
        return ()

    lax.fori_loop(0, PAIRS_PER_WORKER, pair_step, ())


@functools.partial(jax.jit, static_argnames=())
def kernel(x, tables):
    xt_flat = x.astype(jnp.int32).T.reshape(N_FIELDS * BATCH)
    tab_t = tables.transpose(0, 2, 1)

    mesh = plsc.VectorSubcoreMesh(core_axis_name="c", subcore_axis_name="s")
    out5 = pl.kernel(
        _body,
        out_type=jax.ShapeDtypeStruct(
            (N_FIELDS, 2, BATCH // 128, 8, 128), jnp.float32),
        mesh=mesh,
        scratch_types=[
            pltpu.VMEM((VOCAB,), jnp.float32),
            pltpu.VMEM((IDX_CHUNK,), jnp.int32),
            pltpu.VMEM((BATCH // 128, 128), jnp.float32),
            pltpu.SemaphoreType.DMA,
        ],
        compiler_params=pltpu.CompilerParams(use_tc_tiling_on_sc=True,
                                             needs_layout_passes=False),
    )(xt_flat, tab_t)
    return out5.transpose(2, 4, 0, 1, 3).reshape(BATCH, N_FIELDS, EMBED_DIM)


# speedup vs baseline: 9.9452x; 1.5922x over previous
import functools

import jax
import jax.numpy as jnp
from jax import lax
from jax.experimental import pallas as pl
from jax.experimental.pallas import tpu as pltpu
from jax.experimental.pallas import tpu_sc as plsc

N_FIELDS = 26
VOCAB = 100000
EMBED_DIM = 16
BATCH = 16384

NUM_CORES = 2
NUM_SUBCORES = 16
NUM_WORKERS = NUM_CORES * NUM_SUBCORES
PAIRS = N_FIELDS * EMBED_DIM
PAIRS_PER_WORKER = PAIRS // NUM_WORKERS
IDX_CHUNK = 4096
LANES = 16


def _body(xt_hbm, tab_hbm, out_hbm, row_v, idx0_v, idx1_v, ov_v,
          semr, semi0, semi1, semw):
    wid = lax.axis_index("s") * NUM_CORES + lax.axis_index("c")
    p0 = wid * PAIRS_PER_WORKER
    idx_bufs = (idx0_v, idx1_v)
    idx_sems = (semi0, semi1)

    def out_slice(p):
        return out_hbm.at[p >> 4, (p & 15) >> 3, :, p & 7, :]

    def pair_step(k, _):
        p = p0 + k
        f = p >> 4
        cr = pltpu.async_copy(tab_hbm.at[f, p & 15], row_v, semr)
        ci = [pltpu.async_copy(
            xt_hbm.at[pl.ds(f * BATCH + q * IDX_CHUNK, IDX_CHUNK)],
            idx_bufs[q], idx_sems[q], ) for q in range(2)]

        @pl.when(k > 0)
        def _drain():
            pltpu.make_async_copy(ov_v, out_slice(p), semw).wait()

        cr.wait()
        for q in range(BATCH // IDX_CHUNK):
            buf = idx_bufs[q & 1]
            if q < 2:
                ci[q].wait()
            else:
                pltpu.make_async_copy(
                    xt_hbm.at[pl.ds(f * BATCH + q * IDX_CHUNK, IDX_CHUNK)],
                    buf, idx_sems[q & 1]).wait()

            def gat_step(i, _):
                for j in range(8):
                    sl = pl.ds((i * 8 + j) * LANES, LANES)
                    val = plsc.load_gather(row_v, [buf[sl]])
                    ov_v[q * (IDX_CHUNK // 128) + i,
                         pl.ds(j * LANES, LANES)] = val
                return ()

            lax.fori_loop(0, IDX_CHUNK // LANES // 8, gat_step, ())
            if q < 2:
                pltpu.async_copy(
                    xt_hbm.at[pl.ds(f * BATCH + (q + 2) * IDX_CHUNK,
                                    IDX_CHUNK)],
                    buf, idx_sems[q & 1])
        pltpu.async_copy(ov_v, out_slice(p), semw)
        return ()

    lax.fori_loop(0, PAIRS_PER_WORKER, pair_step, ())
    pltpu.make_async_copy(ov_v, out_slice(p0 + PAIRS_PER_WORKER - 1),
                          semw).wait()


@functools.partial(jax.jit, static_argnames=())
def kernel(x, tables):
    xt_flat = x.astype(jnp.int32).T.reshape(N_FIELDS * BATCH)
    tab_t = tables.transpose(0, 2, 1)

    mesh = plsc.VectorSubcoreMesh(core_axis_name="c", subcore_axis_name="s")
    out5 = pl.kernel(
        _body,
        out_type=jax.ShapeDtypeStruct(
            (N_FIELDS, 2, BATCH // 128, 8, 128), jnp.float32),
        mesh=mesh,
        scratch_types=[
            pltpu.VMEM((VOCAB,), jnp.float32),
            pltpu.VMEM((IDX_CHUNK,), jnp.int32),
            pltpu.VMEM((IDX_CHUNK,), jnp.int32),
            pltpu.VMEM((BATCH // 128, 128), jnp.float32),
            pltpu.SemaphoreType.DMA,
            pltpu.SemaphoreType.DMA,
            pltpu.SemaphoreType.DMA,
            pltpu.SemaphoreType.DMA,
        ],
        compiler_params=pltpu.CompilerParams(use_tc_tiling_on_sc=True,
                                             needs_layout_passes=False),
    )(xt_flat, tab_t)
    return out5.transpose(2, 4, 0, 1, 3).reshape(BATCH, N_FIELDS, EMBED_DIM)


# speedup vs baseline: 10.0274x; 1.0083x over previous
import functools

import jax
import jax.numpy as jnp
from jax import lax
from jax.experimental import pallas as pl
from jax.experimental.pallas import tpu as pltpu
from jax.experimental.pallas import tpu_sc as plsc

N_FIELDS = 26
VOCAB = 100000
EMBED_DIM = 16
BATCH = 16384

NUM_CORES = 2
NUM_SUBCORES = 16
NUM_WORKERS = NUM_CORES * NUM_SUBCORES
PAIRS = N_FIELDS * EMBED_DIM
PAIRS_PER_WORKER = PAIRS // NUM_WORKERS
IDX_CHUNK = 4096
LANES = 16


def _body(xt_hbm, tab_hbm, out_hbm, row_v, idx0_v, idx1_v, ov_v,
          semr, semi0, semi1, semw):
    wid = lax.axis_index("s") * NUM_CORES + lax.axis_index("c")
    p0 = wid * PAIRS_PER_WORKER
    idx_bufs = (idx0_v, idx1_v)
    idx_sems = (semi0, semi1)

    def out_slice(p):
        return out_hbm.at[p >> 4, (p & 15) >> 3, :, p & 7, :]

    def pair_step(k, _):
        p = p0 + k
        f = p >> 4
        cr = pltpu.async_copy(tab_hbm.at[f, p & 15], row_v, semr)
        ci = [pltpu.async_copy(
            xt_hbm.at[f, pl.ds(q * IDX_CHUNK, IDX_CHUNK)],
            idx_bufs[q], idx_sems[q], ) for q in range(2)]

        @pl.when(k > 0)
        def _drain():
            pltpu.make_async_copy(ov_v, out_slice(p), semw).wait()

        cr.wait()
        for q in range(BATCH // IDX_CHUNK):
            buf = idx_bufs[q & 1]
            if q < 2:
                ci[q].wait()
            else:
                pltpu.make_async_copy(
                    xt_hbm.at[f, pl.ds(q * IDX_CHUNK, IDX_CHUNK)],
                    buf, idx_sems[q & 1]).wait()

            def gat_step(i, _):
                for j in range(8):
                    sl = pl.ds((i * 8 + j) * LANES, LANES)
                    val = plsc.load_gather(row_v, [buf[sl]])
                    ov_v[q * (IDX_CHUNK // 128) + i,
                         pl.ds(j * LANES, LANES)] = val
                return ()

            lax.fori_loop(0, IDX_CHUNK // LANES // 8, gat_step, ())
            if q < 2:
                pltpu.async_copy(
                    xt_hbm.at[f, pl.ds((q + 2) * IDX_CHUNK, IDX_CHUNK)],
                    buf, idx_sems[q & 1])
        pltpu.async_copy(ov_v, out_slice(p), semw)
        return ()

    lax.fori_loop(0, PAIRS_PER_WORKER, pair_step, ())
    pltpu.make_async_copy(ov_v, out_slice(p0 + PAIRS_PER_WORKER - 1),
                          semw).wait()


@functools.partial(jax.jit, static_argnames=())
def kernel(x, tables):
    xt = x.astype(jnp.int32).T
    tab_t = tables.transpose(0, 2, 1)

    mesh = plsc.VectorSubcoreMesh(core_axis_name="c", subcore_axis_name="s")
    out5 = pl.kernel(
        _body,
        out_type=jax.ShapeDtypeStruct(
            (N_FIELDS, 2, BATCH // 128, 8, 128), jnp.float32),
        mesh=mesh,
        scratch_types=[
            pltpu.VMEM((VOCAB,), jnp.float32),
            pltpu.VMEM((IDX_CHUNK,), jnp.int32),
            pltpu.VMEM((IDX_CHUNK,), jnp.int32),
            pltpu.VMEM((BATCH // 128, 128), jnp.float32),
            pltpu.SemaphoreType.DMA,
            pltpu.SemaphoreType.DMA,
            pltpu.SemaphoreType.DMA,
            pltpu.SemaphoreType.DMA,
        ],
        compiler_params=pltpu.CompilerParams(use_tc_tiling_on_sc=True,
                                             needs_layout_passes=False),
    )(xt, tab_t)
    return out5.transpose(2, 4, 0, 1, 3).reshape(BATCH, N_FIELDS, EMBED_DIM)


# speedup vs baseline: 10.7281x; 1.0699x over previous
import functools

import jax
import jax.numpy as jnp
from jax import lax
from jax.experimental import pallas as pl
from jax.experimental.pallas import tpu as pltpu
from jax.experimental.pallas import tpu_sc as plsc

N_FIELDS = 26
VOCAB = 100000
EMBED_DIM = 16
BATCH = 16384

NUM_CORES = 2
NUM_SUBCORES = 16
NUM_WORKERS = NUM_CORES * NUM_SUBCORES
PAIRS = N_FIELDS * EMBED_DIM
PAIRS_PER_WORKER = PAIRS // NUM_WORKERS
IDX_CHUNK = 4096
LANES = 16


def _body(xt_hbm, tab_hbm, out_hbm, row_v, idx_v, ov0_v, ov1_v,
          semr, semi, semw0, semw1):
    wid = lax.axis_index("s") * NUM_CORES + lax.axis_index("c")
    p0 = wid * PAIRS_PER_WORKER
    ov_bufs = (ov0_v, ov1_v)
    ov_sems = (semw0, semw1)
    QROWS = BATCH // 128 // 4

    def out_slice(p, q):
        return out_hbm.at[p >> 4, (p & 15) >> 3, pl.ds(q * QROWS, QROWS),
                          p & 7, :]

    def pair_step(k, _):
        p = p0 + k
        f = p >> 4
        cr = pltpu.async_copy(tab_hbm.at[f, p & 15], row_v, semr)

        @pl.when((k == 0) | ((p & 15) == 0))
        def _stage_idx():
            pltpu.async_copy(xt_hbm.at[f], idx_v, semi).wait()

        cr.wait()
        for q in range(4):
            buf = ov_bufs[q & 1]
            if q < 2:
                @pl.when(k > 0)
                def _drain():
                    pltpu.make_async_copy(buf, out_slice(p, q + 2),
                                          ov_sems[q & 1]).wait()
            else:
                pltpu.make_async_copy(buf, out_slice(p, q - 2),
                                      ov_sems[q & 1]).wait()

            def gat_step(i, _):
                for j in range(8):
                    sl = pl.ds(((q * QROWS + i) * 8 + j) * LANES, LANES)
                    val = plsc.load_gather(row_v, [idx_v[sl]])
                    buf[i, pl.ds(j * LANES, LANES)] = val
                return ()

            lax.fori_loop(0, QROWS, gat_step, ())
            pltpu.async_copy(buf, out_slice(p, q), ov_sems[q & 1])
        return ()

    lax.fori_loop(0, PAIRS_PER_WORKER, pair_step, ())
    for q in (2, 3):
        pltpu.make_async_copy(ov_bufs[q & 1],
                              out_slice(p0 + PAIRS_PER_WORKER - 1, q),
                              ov_sems[q & 1]).wait()


@functools.partial(jax.jit, static_argnames=())
def kernel(x, tables):
    xt = x.astype(jnp.int32).T
    tab_t = tables.transpose(0, 2, 1)

    mesh = plsc.VectorSubcoreMesh(core_axis_name="c", subcore_axis_name="s")
    out5 = pl.kernel(
        _body,
        out_type=jax.ShapeDtypeStruct(
            (N_FIELDS, 2, BATCH // 128, 8, 128), jnp.float32),
        mesh=mesh,
        scratch_types=[
            pltpu.VMEM((VOCAB,), jnp.float32),
            pltpu.VMEM((BATCH,), jnp.int32),
            pltpu.VMEM((BATCH // 128 // 4, 128), jnp.float32),
            pltpu.VMEM((BATCH // 128 // 4, 128), jnp.float32),
            pltpu.SemaphoreType.DMA,
            pltpu.SemaphoreType.DMA,
            pltpu.SemaphoreType.DMA,
            pltpu.SemaphoreType.DMA,
        ],
        compiler_params=pltpu.CompilerParams(use_tc_tiling_on_sc=True,
                                             needs_layout_passes=False),
    )(xt, tab_t)
    return out5.transpose(2, 4, 0, 1, 3).reshape(BATCH, N_FIELDS, EMBED_DIM)
